# Initial kernel scaffold; baseline (speedup 1.0000x reference)
import jax
import jax.numpy as jnp
from jax.experimental import pallas as pl


def kernel(user_emb, item_emb, edge_index, edge_weight):
    raise NotImplementedError("write your pallas kernel here")


# speedup vs baseline: 2.7523x; 2.7523x over previous
import functools

import jax
import jax.numpy as jnp
from jax import lax
from jax.experimental import pallas as pl
from jax.experimental.pallas import tpu as pltpu
from jax.experimental.pallas import tpu_sc as plsc

_NUM_USERS = 20000
_NUM_ITEMS = 30000
_N = _NUM_USERS + _NUM_ITEMS
_D = 64
_E = 800000
_HALF = _N // 2
_CHUNK = 128
_NCHUNKS = _E // _CHUNK
_NS = 16
_ROW_CHUNKS = (_HALF + _CHUNK) // _CHUNK + 1
_ACC_ROWS = _ROW_CHUNKS * _CHUNK
_FULL_OUT_CHUNKS = _HALF // _CHUNK
_OUT_REM = _HALF - _FULL_OUT_CHUNKS * _CHUNK


_GATHER_DNUMS = lax.GatherDimensionNumbers(
    offset_dims=(), collapsed_slice_dims=(0,), start_index_map=(0,))


def _lane_bcast(vec, j):
    idx = jnp.full((16, 1), j, dtype=jnp.int32)
    return lax.gather(vec, idx, _GATHER_DNUMS, (1,),
                      mode=lax.GatherScatterMode.PROMISE_IN_BOUNDS)


def _spmm_body(x_hbm, src_hbm, dst_hbm, w_hbm, out_hbm,
               src_idx, dst_idx, w_v, rows, acc, sem):
    c = lax.axis_index("c")
    s = lax.axis_index("s")
    base = c * _HALF

    def _zero_rows(r, carry):
        for dd in range(4):
            rows[r, pl.ds(dd * 16, 16)] = jnp.zeros((16,), jnp.float32)
        return carry
    lax.fori_loop(0, _CHUNK, _zero_rows, 0)

    nz = (_ROW_CHUNKS - s + _NS - 1) // _NS
    def _zero_acc(i, carry):
        ct = s + _NS * i
        pltpu.sync_copy(rows, acc.at[pl.ds(ct * _CHUNK, _CHUNK)])
        return carry
    lax.fori_loop(0, nz, _zero_acc, 0)

    plsc.subcore_barrier()

    ne = (_NCHUNKS - s + _NS - 1) // _NS
    def _edge_chunk(i, carry):
        e0 = (s + _NS * i) * _CHUNK
        pltpu.sync_copy(src_hbm.at[pl.ds(e0, _CHUNK)], src_idx)
        pltpu.sync_copy(dst_hbm.at[pl.ds(e0, _CHUNK)], dst_idx)
        pltpu.sync_copy(w_hbm.at[pl.ds(e0, _CHUNK)], w_v)
        pltpu.async_copy(x_hbm.at[src_idx], rows, sem).wait()
        for g in range(8):
            sl = pl.ds(g * 16, 16)
            loc = dst_idx[sl] - base
            ok = (loc >= 0) & (loc < _HALF)
            dst_idx[sl] = jnp.where(ok, loc, _HALF)
        for g in range(8):
            wv = w_v[pl.ds(g * 16, 16)]
            for j in range(16):
                r = g * 16 + j
                wb = _lane_bcast(wv, j)
                for dd in range(4):
                    sl = pl.ds(dd * 16, 16)
                    rows[r, sl] = rows[r, sl] * wb
        pltpu.sync_copy(rows, acc.at[dst_idx], add=True)
        return carry
    lax.fori_loop(0, ne, _edge_chunk, 0)

    plsc.subcore_barrier()

    no = (_FULL_OUT_CHUNKS - s + _NS - 1) // _NS
    def _out_chunk(i, carry):
        ct = s + _NS * i
        pltpu.sync_copy(acc.at[pl.ds(ct * _CHUNK, _CHUNK)], rows)
        pltpu.sync_copy(rows, out_hbm.at[pl.ds(base + ct * _CHUNK, _CHUNK)])
        return carry
    lax.fori_loop(0, no, _out_chunk, 0)

    @pl.when(s == _FULL_OUT_CHUNKS % _NS)
    def _():
        r0 = _FULL_OUT_CHUNKS * _CHUNK
        pltpu.sync_copy(acc.at[pl.ds(r0, _OUT_REM)], rows.at[pl.ds(0, _OUT_REM)])
        pltpu.sync_copy(rows.at[pl.ds(0, _OUT_REM)],
                        out_hbm.at[pl.ds(base + r0, _OUT_REM)])


_spmm = functools.partial(
    pl.kernel,
    out_type=jax.ShapeDtypeStruct((_N, _D), jnp.float32),
    mesh=plsc.VectorSubcoreMesh(core_axis_name="c", subcore_axis_name="s"),
    compiler_params=pltpu.CompilerParams(use_tc_tiling_on_sc=False),
    scratch_types=[
        pltpu.VMEM((_CHUNK,), jnp.int32),
        pltpu.VMEM((_CHUNK,), jnp.int32),
        pltpu.VMEM((_CHUNK,), jnp.float32),
        pltpu.VMEM((_CHUNK, _D), jnp.float32),
        pltpu.VMEM_SHARED((_ACC_ROWS, _D), jnp.float32),
        pltpu.SemaphoreType.DMA,
    ],
)(_spmm_body)


def _combine_body(a_ref, b_ref, c_ref, d_ref, o_ref):
    o_ref[...] = 0.25 * (a_ref[...] + b_ref[...] + c_ref[...] + d_ref[...])


_combine = pl.pallas_call(
    _combine_body,
    out_shape=jax.ShapeDtypeStruct((_N, _D), jnp.float32),
    grid=(50,),
    in_specs=[pl.BlockSpec((1000, _D), lambda i: (i, 0))] * 4,
    out_specs=pl.BlockSpec((1000, _D), lambda i: (i, 0)),
)


def kernel(user_emb, item_emb, edge_index, edge_weight):
    x0 = jnp.concatenate([user_emb, item_emb], axis=0)
    src = edge_index[0]
    dst = edge_index[1]
    x1 = _spmm(x0, src, dst, edge_weight)
    x2 = _spmm(x1, src, dst, edge_weight)
    x3 = _spmm(x2, src, dst, edge_weight)
    x4 = _spmm(x3, src, dst, edge_weight)
    light = _combine(x0, x2, x3, x4)
    return light[:_NUM_USERS], light[_NUM_USERS:]


# speedup vs baseline: 3.1710x; 1.1521x over previous
import functools

import jax
import jax.numpy as jnp
from jax import lax
from jax.experimental import pallas as pl
from jax.experimental.pallas import tpu as pltpu
from jax.experimental.pallas import tpu_sc as plsc

_NUM_USERS = 20000
_NUM_ITEMS = 30000
_N = _NUM_USERS + _NUM_ITEMS
_D = 64
_DH = _D // 2
_E = 800000
_CHUNK = 128
_NCHUNKS = _E // _CHUNK
_NS = 16
_ROW_CHUNKS = -(-_N // _CHUNK)
_ACC_ROWS = _ROW_CHUNKS * _CHUNK
_FULL_OUT_CHUNKS = _N // _CHUNK
_OUT_REM = _N - _FULL_OUT_CHUNKS * _CHUNK

_GATHER_DNUMS = lax.GatherDimensionNumbers(
    offset_dims=(), collapsed_slice_dims=(0,), start_index_map=(0,))


def _lane_bcast(vec, j):
    idx = jnp.full((16, 1), j, dtype=jnp.int32)
    return lax.gather(vec, idx, _GATHER_DNUMS, (1,),
                      mode=lax.GatherScatterMode.PROMISE_IN_BOUNDS)


def _spmm_body(xlo_hbm, xhi_hbm, src_hbm, dst_hbm, w_hbm,
               outlo_hbm, outhi_hbm,
               src_idx, dst_idx, w_v, rows, acc, sem):
    c = lax.axis_index("c")
    s = lax.axis_index("s")

    def _zero_rows(r, carry):
        for dd in range(_DH // 16):
            rows[r, pl.ds(dd * 16, 16)] = jnp.zeros((16,), jnp.float32)
        return carry
    lax.fori_loop(0, _CHUNK, _zero_rows, 0)

    nz = (_ROW_CHUNKS - s + _NS - 1) // _NS
    def _zero_acc(i, carry):
        ct = s + _NS * i
        pltpu.sync_copy(rows, acc.at[pl.ds(ct * _CHUNK, _CHUNK)])
        return carry
    lax.fori_loop(0, nz, _zero_acc, 0)

    plsc.subcore_barrier()

    ne = (_NCHUNKS - s + _NS - 1) // _NS
    def _edge_chunk(i, carry):
        e0 = (s + _NS * i) * _CHUNK
        pltpu.sync_copy(src_hbm.at[pl.ds(e0, _CHUNK)], src_idx)
        pltpu.sync_copy(dst_hbm.at[pl.ds(e0, _CHUNK)], dst_idx)
        pltpu.sync_copy(w_hbm.at[pl.ds(e0, _CHUNK)], w_v)

        @pl.when(c == 0)
        def _():
            pltpu.async_copy(xlo_hbm.at[src_idx], rows, sem).wait()

        @pl.when(c == 1)
        def _():
            pltpu.async_copy(xhi_hbm.at[src_idx], rows, sem).wait()

        for g in range(_CHUNK // 16):
            wv = w_v[pl.ds(g * 16, 16)]
            for j in range(16):
                r = g * 16 + j
                wb = _lane_bcast(wv, j)
                for dd in range(_DH // 16):
                    sl = pl.ds(dd * 16, 16)
                    rows[r, sl] = rows[r, sl] * wb

        pltpu.sync_copy(rows, acc.at[dst_idx], add=True)
        return carry
    lax.fori_loop(0, ne, _edge_chunk, 0)

    plsc.subcore_barrier()

    no = (_FULL_OUT_CHUNKS - s + _NS - 1) // _NS
    def _out_chunk(i, carry):
        ct = s + _NS * i
        pltpu.sync_copy(acc.at[pl.ds(ct * _CHUNK, _CHUNK)], rows)

        @pl.when(c == 0)
        def _():
            pltpu.sync_copy(rows, outlo_hbm.at[pl.ds(ct * _CHUNK, _CHUNK)])

        @pl.when(c == 1)
        def _():
            pltpu.sync_copy(rows, outhi_hbm.at[pl.ds(ct * _CHUNK, _CHUNK)])
        return carry
    lax.fori_loop(0, no, _out_chunk, 0)

    @pl.when(s == _FULL_OUT_CHUNKS % _NS)
    def _():
        r0 = _FULL_OUT_CHUNKS * _CHUNK
        pltpu.sync_copy(acc.at[pl.ds(r0, _OUT_REM)], rows.at[pl.ds(0, _OUT_REM)])

        @pl.when(c == 0)
        def _():
            pltpu.sync_copy(rows.at[pl.ds(0, _OUT_REM)],
                            outlo_hbm.at[pl.ds(r0, _OUT_REM)])

        @pl.when(c == 1)
        def _():
            pltpu.sync_copy(rows.at[pl.ds(0, _OUT_REM)],
                            outhi_hbm.at[pl.ds(r0, _OUT_REM)])


_spmm = functools.partial(
    pl.kernel,
    out_type=(
        jax.ShapeDtypeStruct((_N, _DH), jnp.float32),
        jax.ShapeDtypeStruct((_N, _DH), jnp.float32),
    ),
    mesh=plsc.VectorSubcoreMesh(core_axis_name="c", subcore_axis_name="s"),
    compiler_params=pltpu.CompilerParams(use_tc_tiling_on_sc=False),
    scratch_types=[
        pltpu.VMEM((_CHUNK,), jnp.int32),
        pltpu.VMEM((_CHUNK,), jnp.int32),
        pltpu.VMEM((_CHUNK,), jnp.float32),
        pltpu.VMEM((_CHUNK, _DH), jnp.float32),
        pltpu.VMEM_SHARED((_ACC_ROWS, _DH), jnp.float32),
        pltpu.SemaphoreType.DMA,
    ],
)(_spmm_body)


def _combine_body(x0_ref, lo2, hi2, lo3, hi3, lo4, hi4, o_ref):
    lo = lo2[...] + lo3[...] + lo4[...]
    hi = hi2[...] + hi3[...] + hi4[...]
    o_ref[...] = 0.25 * (x0_ref[...] + jnp.concatenate([lo, hi], axis=1))


_combine = pl.pallas_call(
    _combine_body,
    out_shape=jax.ShapeDtypeStruct((_N, _D), jnp.float32),
    grid=(50,),
    in_specs=[pl.BlockSpec((1000, _D), lambda i: (i, 0))]
    + [pl.BlockSpec((1000, _DH), lambda i: (i, 0))] * 6,
    out_specs=pl.BlockSpec((1000, _D), lambda i: (i, 0)),
)


def kernel(user_emb, item_emb, edge_index, edge_weight):
    x0 = jnp.concatenate([user_emb, item_emb], axis=0)
    src = edge_index[0]
    dst = edge_index[1]
    lo0, hi0 = x0[:, :_DH], x0[:, _DH:]
    lo1, hi1 = _spmm(lo0, hi0, src, dst, edge_weight)
    lo2, hi2 = _spmm(lo1, hi1, src, dst, edge_weight)
    lo3, hi3 = _spmm(lo2, hi2, src, dst, edge_weight)
    lo4, hi4 = _spmm(lo3, hi3, src, dst, edge_weight)
    light = _combine(x0, lo2, hi2, lo3, hi3, lo4, hi4)
    return light[:_NUM_USERS], light[_NUM_USERS:]


# speedup vs baseline: 6.9079x; 2.1784x over previous
import functools

import jax
import jax.numpy as jnp
from jax import lax
from jax.experimental import pallas as pl
from jax.experimental.pallas import tpu as pltpu
from jax.experimental.pallas import tpu_sc as plsc

_NUM_USERS = 20000
_NUM_ITEMS = 30000
_N = _NUM_USERS + _NUM_ITEMS
_D = 64
_DH = _D // 2
_E = 800000
_CHUNK = 128
_NCHUNKS = _E // _CHUNK
_IB = 10
_NG = _NCHUNKS // _IB
_PAIRS = _IB // 2
_NS = 16
_ROW_CHUNKS = -(-_N // _CHUNK)
_ACC_ROWS = _ROW_CHUNKS * _CHUNK
_FULL_OUT_CHUNKS = _N // _CHUNK
_OUT_REM = _N - _FULL_OUT_CHUNKS * _CHUNK

_GATHER_DNUMS = lax.GatherDimensionNumbers(
    offset_dims=(), collapsed_slice_dims=(0,), start_index_map=(0,))


def _lane_bcast(vec, j):
    idx = jnp.full((16, 1), j, dtype=jnp.int32)
    return lax.gather(vec, idx, _GATHER_DNUMS, (1,),
                      mode=lax.GatherScatterMode.PROMISE_IN_BOUNDS)


def _scale_rows(rows, wvb, ch):
    for g in range(_CHUNK // 16):
        wv = wvb[ch, pl.ds(g * 16, 16)]
        for j in range(16):
            r = g * 16 + j
            wb = _lane_bcast(wv, j)
            for dd in range(_DH // 16):
                sl = pl.ds(dd * 16, 16)
                rows[r, sl] = rows[r, sl] * wb


def _edge_phase(x_hbm, src2d, dst2d, w2d, srcb, dstb, wvb,
                rows0, rows1, acc, gsem, s):
    ng = (_NG - s + _NS - 1) // _NS

    def _group(i, carry):
        c0 = (s + _NS * i) * _IB
        pltpu.sync_copy(src2d.at[pl.ds(c0, _IB)], srcb)
        pltpu.sync_copy(dst2d.at[pl.ds(c0, _IB)], dstb)
        pltpu.sync_copy(w2d.at[pl.ds(c0, _IB)], wvb)
        pltpu.async_copy(x_hbm.at[srcb.at[0]], rows0, gsem)

        def _pair(p, carry2):
            a = 2 * p
            b = a + 1
            pltpu.make_async_copy(x_hbm.at[srcb.at[a]], rows0, gsem).wait()
            pltpu.async_copy(x_hbm.at[srcb.at[b]], rows1, gsem)
            _scale_rows(rows0, wvb, a)
            pltpu.sync_copy(rows0, acc.at[dstb.at[a]], add=True)
            pltpu.make_async_copy(x_hbm.at[srcb.at[b]], rows1, gsem).wait()

            @pl.when(p < _PAIRS - 1)
            def _():
                pltpu.async_copy(x_hbm.at[srcb.at[a + 2]], rows0, gsem)

            _scale_rows(rows1, wvb, b)
            pltpu.sync_copy(rows1, acc.at[dstb.at[b]], add=True)
            return carry2
        lax.fori_loop(0, _PAIRS, _pair, 0)
        return carry
    lax.fori_loop(0, ng, _group, 0)


def _spmm_body(xlo_hbm, xhi_hbm, src2d, dst2d, w2d,
               outlo_hbm, outhi_hbm,
               srcb, dstb, wvb, rows0, rows1, acc, gsem):
    c = lax.axis_index("c")
    s = lax.axis_index("s")

    def _zero_rows(r, carry):
        for dd in range(_DH // 16):
            rows0[r, pl.ds(dd * 16, 16)] = jnp.zeros((16,), jnp.float32)
        return carry
    lax.fori_loop(0, _CHUNK, _zero_rows, 0)

    nz = (_ROW_CHUNKS - s + _NS - 1) // _NS
    def _zero_acc(i, carry):
        ct = s + _NS * i
        pltpu.sync_copy(rows0, acc.at[pl.ds(ct * _CHUNK, _CHUNK)])
        return carry
    lax.fori_loop(0, nz, _zero_acc, 0)

    plsc.subcore_barrier()

    @pl.when(c == 0)
    def _():
        _edge_phase(xlo_hbm, src2d, dst2d, w2d, srcb, dstb, wvb,
                    rows0, rows1, acc, gsem, s)

    @pl.when(c == 1)
    def _():
        _edge_phase(xhi_hbm, src2d, dst2d, w2d, srcb, dstb, wvb,
                    rows0, rows1, acc, gsem, s)

    plsc.subcore_barrier()

    no = (_FULL_OUT_CHUNKS - s + _NS - 1) // _NS
    def _out_chunk(i, carry):
        r0 = (s + _NS * i) * _CHUNK
        sl = pl.ds(r0, _CHUNK)

        @pl.when(c == 0)
        def _():
            pltpu.sync_copy(acc.at[sl], outlo_hbm.at[sl])

        @pl.when(c == 1)
        def _():
            pltpu.sync_copy(acc.at[sl], outhi_hbm.at[sl])
        return carry
    lax.fori_loop(0, no, _out_chunk, 0)

    @pl.when(s == _FULL_OUT_CHUNKS % _NS)
    def _():
        sl = pl.ds(_FULL_OUT_CHUNKS * _CHUNK, _OUT_REM)

        @pl.when(c == 0)
        def _():
            pltpu.sync_copy(acc.at[sl], outlo_hbm.at[sl])

        @pl.when(c == 1)
        def _():
            pltpu.sync_copy(acc.at[sl], outhi_hbm.at[sl])


_spmm = functools.partial(
    pl.kernel,
    out_type=(
        jax.ShapeDtypeStruct((_N, _DH), jnp.float32),
        jax.ShapeDtypeStruct((_N, _DH), jnp.float32),
    ),
    mesh=plsc.VectorSubcoreMesh(core_axis_name="c", subcore_axis_name="s"),
    compiler_params=pltpu.CompilerParams(use_tc_tiling_on_sc=False),
    scratch_types=[
        pltpu.VMEM((_IB, _CHUNK), jnp.int32),
        pltpu.VMEM((_IB, _CHUNK), jnp.int32),
        pltpu.VMEM((_IB, _CHUNK), jnp.float32),
        pltpu.VMEM((_CHUNK, _DH), jnp.float32),
        pltpu.VMEM((_CHUNK, _DH), jnp.float32),
        pltpu.VMEM_SHARED((_ACC_ROWS, _DH), jnp.float32),
        pltpu.SemaphoreType.DMA,
    ],
)(_spmm_body)


def _combine_body(x0_ref, lo2, hi2, lo3, hi3, lo4, hi4, o_ref):
    lo = lo2[...] + lo3[...] + lo4[...]
    hi = hi2[...] + hi3[...] + hi4[...]
    o_ref[...] = 0.25 * (x0_ref[...] + jnp.concatenate([lo, hi], axis=1))


_combine = pl.pallas_call(
    _combine_body,
    out_shape=jax.ShapeDtypeStruct((_N, _D), jnp.float32),
    grid=(50,),
    in_specs=[pl.BlockSpec((1000, _D), lambda i: (i, 0))]
    + [pl.BlockSpec((1000, _DH), lambda i: (i, 0))] * 6,
    out_specs=pl.BlockSpec((1000, _D), lambda i: (i, 0)),
)


def kernel(user_emb, item_emb, edge_index, edge_weight):
    x0 = jnp.concatenate([user_emb, item_emb], axis=0)
    src2d = edge_index[0].reshape(_NCHUNKS, _CHUNK)
    dst2d = edge_index[1].reshape(_NCHUNKS, _CHUNK)
    w2d = edge_weight.reshape(_NCHUNKS, _CHUNK)
    lo0, hi0 = x0[:, :_DH], x0[:, _DH:]
    lo1, hi1 = _spmm(lo0, hi0, src2d, dst2d, w2d)
    lo2, hi2 = _spmm(lo1, hi1, src2d, dst2d, w2d)
    lo3, hi3 = _spmm(lo2, hi2, src2d, dst2d, w2d)
    lo4, hi4 = _spmm(lo3, hi3, src2d, dst2d, w2d)
    light = _combine(x0, lo2, hi2, lo3, hi3, lo4, hi4)
    return light[:_NUM_USERS], light[_NUM_USERS:]
